# initial kernel scaffold (unmeasured)
import functools

import jax
import jax.numpy as jnp
from jax import lax
from jax.experimental import pallas as pl
from jax.experimental.pallas import tpu as pltpu

B = 8
H = 8
D = 64
BS = 16
NB = 64
NPAGES_LOCAL = 64
HD = H * D
ROWS = NPAGES_LOCAL * BS
SCALE = D ** -0.5


def _body(q_ref, k_ref, v_ref, bt_ref, lens_ref, out_ref,
          kf, vf, ks, vs, send_sems, recv_sems):
    my_x = lax.axis_index("x")
    my_y = lax.axis_index("y")
    my_z = lax.axis_index("z")
    nbr = (1 - my_x, my_y, my_z)

    barrier_sem = pltpu.get_barrier_semaphore()
    pl.semaphore_signal(barrier_sem, inc=1, device_id=nbr,
                        device_id_type=pl.DeviceIdType.MESH)
    pl.semaphore_wait(barrier_sem, 1)

    rk = pltpu.make_async_remote_copy(
        src_ref=k_ref,
        dst_ref=kf.at[pl.ds(ROWS, ROWS), :],
        send_sem=send_sems.at[0],
        recv_sem=recv_sems.at[0],
        device_id=nbr,
        device_id_type=pl.DeviceIdType.MESH,
    )
    rv = pltpu.make_async_remote_copy(
        src_ref=v_ref,
        dst_ref=vf.at[pl.ds(ROWS, ROWS), :],
        send_sem=send_sems.at[1],
        recv_sem=recv_sems.at[1],
        device_id=nbr,
        device_id_type=pl.DeviceIdType.MESH,
    )
    rk.start()
    rv.start()

    kf[pl.ds(0, ROWS), :] = k_ref[:, :]
    vf[pl.ds(0, ROWS), :] = v_ref[:, :]

    rk.wait()
    rv.wait()

    kidx = lax.broadcasted_iota(jnp.int32, (1, NB * BS), 1)

    def per_batch(i, carry):
        def gather_one(j, c):
            p = bt_ref[i, j]
            off = (jnp.where(p // NPAGES_LOCAL == my_x, 0, ROWS)
                   + (p % NPAGES_LOCAL) * BS)
            ks[pl.ds(j * BS, BS), :] = kf[pl.ds(off, BS), :]
            vs[pl.ds(j * BS, BS), :] = vf[pl.ds(off, BS), :]
            return c
        lax.fori_loop(0, NB, gather_one, 0)

        n_valid = lens_ref[i] * BS
        mask = kidx < n_valid
        qrow = q_ref[pl.ds(i, 1), :]
        for h in range(H):
            qh = qrow[:, h * D:(h + 1) * D]
            ksh = ks[:, h * D:(h + 1) * D]
            s = lax.dot_general(
                qh, ksh, (((1,), (1,)), ((), ())),
                preferred_element_type=jnp.float32,
            ) * SCALE
            s = jnp.where(mask, s, -1e30)
            m = jnp.max(s, axis=1, keepdims=True)
            p_ = jnp.exp(s - m)
            denom = jnp.sum(p_, axis=1, keepdims=True)
            vsh = vs[:, h * D:(h + 1) * D]
            o = lax.dot_general(
                p_, vsh, (((1,), (0,)), ((), ())),
                preferred_element_type=jnp.float32,
            )
            out_ref[pl.ds(i, 1), h * D:(h + 1) * D] = o / denom
        return carry

    lax.fori_loop(0, B, per_batch, 0)

    @functools.partial(pl.run_scoped, exit_sem=pltpu.SemaphoreType.REGULAR)
    def _(exit_sem):
        pl.semaphore_signal(exit_sem, inc=1, device_id=nbr,
                            device_id_type=pl.DeviceIdType.MESH)
        pl.semaphore_wait(exit_sem, 1)


def kernel(Q, K, V, bt, lens):
    q2 = Q.reshape(B, HD)
    k2 = K.reshape(ROWS, HD)
    v2 = V.reshape(ROWS, HD)

    out2 = pl.pallas_call(
        _body,
        out_shape=jax.ShapeDtypeStruct((B, HD), jnp.float32),
        in_specs=[
            pl.BlockSpec(memory_space=pltpu.VMEM),
            pl.BlockSpec(memory_space=pltpu.VMEM),
            pl.BlockSpec(memory_space=pltpu.VMEM),
            pl.BlockSpec(memory_space=pltpu.SMEM),
            pl.BlockSpec(memory_space=pltpu.SMEM),
        ],
        out_specs=pl.BlockSpec(memory_space=pltpu.VMEM),
        scratch_shapes=[
            pltpu.VMEM((2 * ROWS, HD), jnp.float32),
            pltpu.VMEM((2 * ROWS, HD), jnp.float32),
            pltpu.VMEM((NB * BS, HD), jnp.float32),
            pltpu.VMEM((NB * BS, HD), jnp.float32),
            pltpu.SemaphoreType.DMA((2,)),
            pltpu.SemaphoreType.DMA((2,)),
        ],
        compiler_params=pltpu.CompilerParams(collective_id=0),
    )(q2, k2, v2, bt, lens)
    return out2.reshape(B, 1, H, D)


# baseline (device time: 87763 ns/iter reference)
import functools

import jax
import jax.numpy as jnp
from jax import lax
from jax.experimental import pallas as pl
from jax.experimental.pallas import tpu as pltpu

B = 8
H = 8
D = 64
BS = 16
NB = 64
NPAGES_LOCAL = 64
HD = H * D
ROWS = NPAGES_LOCAL * BS
SCALE = D ** -0.5


def _body(q_ref, k_ref, v_ref, bt_ref, lens_ref, out_ref,
          kf, vf, ks, vs, send_sems, recv_sems):
    my_x = lax.axis_index("x")
    my_y = lax.axis_index("y")
    my_z = lax.axis_index("z")
    nbr = (1 - my_x, my_y, my_z)

    barrier_sem = pltpu.get_barrier_semaphore()
    pl.semaphore_signal(barrier_sem, inc=1, device_id=nbr,
                        device_id_type=pl.DeviceIdType.MESH)
    pl.semaphore_wait(barrier_sem, 1)

    rk = pltpu.make_async_remote_copy(
        src_ref=k_ref,
        dst_ref=kf.at[pl.ds(ROWS, ROWS), :],
        send_sem=send_sems.at[0],
        recv_sem=recv_sems.at[0],
        device_id=nbr,
        device_id_type=pl.DeviceIdType.MESH,
    )
    rv = pltpu.make_async_remote_copy(
        src_ref=v_ref,
        dst_ref=vf.at[pl.ds(ROWS, ROWS), :],
        send_sem=send_sems.at[1],
        recv_sem=recv_sems.at[1],
        device_id=nbr,
        device_id_type=pl.DeviceIdType.MESH,
    )
    rk.start()
    rv.start()

    kf[pl.ds(0, ROWS), :] = k_ref[:, :]
    vf[pl.ds(0, ROWS), :] = v_ref[:, :]

    rk.wait()
    rv.wait()

    kidx = lax.broadcasted_iota(jnp.int32, (1, NB * BS), 1)

    def per_batch(i, carry):
        def gather_one(j, c):
            p = bt_ref[i, j]
            off = (jnp.where(p // NPAGES_LOCAL == my_x, 0, ROWS)
                   + (p % NPAGES_LOCAL) * BS)
            ks[pl.ds(j * BS, BS), :] = kf[pl.ds(off, BS), :]
            vs[pl.ds(j * BS, BS), :] = vf[pl.ds(off, BS), :]
            return c
        lax.fori_loop(0, NB, gather_one, 0)

        n_valid = lens_ref[i] * BS
        mask = kidx < n_valid
        qrow = q_ref[pl.ds(i, 1), :]
        outs = []
        for h in range(H):
            qh = qrow[:, h * D:(h + 1) * D]
            ksh = ks[:, h * D:(h + 1) * D]
            s = lax.dot_general(
                qh, ksh, (((1,), (1,)), ((), ())),
                preferred_element_type=jnp.float32,
            ) * SCALE
            s = jnp.where(mask, s, -1e30)
            m = jnp.max(s, axis=1, keepdims=True)
            p_ = jnp.exp(s - m)
            denom = jnp.sum(p_, axis=1, keepdims=True)
            vsh = vs[:, h * D:(h + 1) * D]
            o = lax.dot_general(
                p_, vsh, (((1,), (0,)), ((), ())),
                preferred_element_type=jnp.float32,
            )
            outs.append(o / denom)
        out_ref[pl.ds(i, 1), :] = jnp.concatenate(outs, axis=1)
        return carry

    lax.fori_loop(0, B, per_batch, 0)

    @functools.partial(pl.run_scoped, exit_sem=pltpu.SemaphoreType.REGULAR)
    def _(exit_sem):
        pl.semaphore_signal(exit_sem, inc=1, device_id=nbr,
                            device_id_type=pl.DeviceIdType.MESH)
        pl.semaphore_wait(exit_sem, 1)


def kernel(Q, K, V, bt, lens):
    q2 = Q.reshape(B, HD)
    k2 = K.reshape(ROWS, HD)
    v2 = V.reshape(ROWS, HD)

    out2 = pl.pallas_call(
        _body,
        out_shape=jax.ShapeDtypeStruct((B, HD), jnp.float32),
        in_specs=[
            pl.BlockSpec(memory_space=pltpu.VMEM),
            pl.BlockSpec(memory_space=pltpu.VMEM),
            pl.BlockSpec(memory_space=pltpu.VMEM),
            pl.BlockSpec(memory_space=pltpu.SMEM),
            pl.BlockSpec(memory_space=pltpu.SMEM),
        ],
        out_specs=pl.BlockSpec(memory_space=pltpu.VMEM),
        scratch_shapes=[
            pltpu.VMEM((2 * ROWS, HD), jnp.float32),
            pltpu.VMEM((2 * ROWS, HD), jnp.float32),
            pltpu.VMEM((NB * BS, HD), jnp.float32),
            pltpu.VMEM((NB * BS, HD), jnp.float32),
            pltpu.SemaphoreType.DMA((2,)),
            pltpu.SemaphoreType.DMA((2,)),
        ],
        compiler_params=pltpu.CompilerParams(collective_id=0),
    )(q2, k2, v2, bt, lens)
    return out2.reshape(B, 1, H, D)


# device time: 52079 ns/iter; 1.6852x vs baseline; 1.6852x over previous
import functools

import jax
import jax.numpy as jnp
from jax import lax
from jax.experimental import pallas as pl
from jax.experimental.pallas import tpu as pltpu

B = 8
H = 8
D = 64
BS = 16
NB = 64
NPAGES_LOCAL = 64
HD = H * D
ROWS = NPAGES_LOCAL * BS
SCALE = D ** -0.5


def _body(k_ref, v_ref, qmat_ref, bt_ref, lens_ref, out_ref,
          kf, vf, ks, vs, send_sems, recv_sems):
    my_x = lax.axis_index("x")
    my_y = lax.axis_index("y")
    my_z = lax.axis_index("z")
    nbr = (1 - my_x, my_y, my_z)

    kf[pl.ds(0, ROWS), :] = k_ref[:, :].astype(jnp.bfloat16)
    vf[pl.ds(0, ROWS), :] = v_ref[:, :].astype(jnp.bfloat16)

    barrier_sem = pltpu.get_barrier_semaphore()
    pl.semaphore_signal(barrier_sem, inc=1, device_id=nbr,
                        device_id_type=pl.DeviceIdType.MESH)
    pl.semaphore_wait(barrier_sem, 1)

    rk = pltpu.make_async_remote_copy(
        src_ref=kf.at[pl.ds(0, ROWS), :],
        dst_ref=kf.at[pl.ds(ROWS, ROWS), :],
        send_sem=send_sems.at[0],
        recv_sem=recv_sems.at[0],
        device_id=nbr,
        device_id_type=pl.DeviceIdType.MESH,
    )
    rv = pltpu.make_async_remote_copy(
        src_ref=vf.at[pl.ds(0, ROWS), :],
        dst_ref=vf.at[pl.ds(ROWS, ROWS), :],
        send_sem=send_sems.at[1],
        recv_sem=recv_sems.at[1],
        device_id=nbr,
        device_id_type=pl.DeviceIdType.MESH,
    )
    rk.start()
    rv.start()
    rk.wait()
    rv.wait()

    kmask = lax.broadcasted_iota(jnp.int32, (NB * BS, 1), 0)

    def per_batch(i, carry):
        def gather_one(j, c):
            p = bt_ref[i, j]
            off = (jnp.where(p // NPAGES_LOCAL == my_x, 0, ROWS)
                   + (p % NPAGES_LOCAL) * BS)
            ks[pl.ds(j * BS, BS), :] = kf[pl.ds(off, BS), :]
            vs[pl.ds(j * BS, BS), :] = vf[pl.ds(off, BS), :]
            return c
        lax.fori_loop(0, NB, gather_one, 0)

        n_valid = lens_ref[i] * BS
        qmat = qmat_ref[pl.ds(i * HD, HD), :]
        s = lax.dot_general(
            ks[:, :], qmat, (((1,), (0,)), ((), ())),
            preferred_element_type=jnp.float32,
        ) * SCALE
        s = jnp.where(kmask < n_valid, s, -1e30)
        m = jnp.max(s, axis=0, keepdims=True)
        p_ = jnp.exp(s - m)
        denom = jnp.sum(p_, axis=0, keepdims=True)
        r = lax.dot_general(
            p_.astype(jnp.bfloat16), vs[:, :], (((0,), (0,)), ((), ())),
            preferred_element_type=jnp.float32,
        )
        outs = [
            r[h:h + 1, h * D:(h + 1) * D] / denom[0:1, h:h + 1]
            for h in range(H)
        ]
        out_ref[pl.ds(i, 1), :] = jnp.concatenate(outs, axis=1)
        return carry

    lax.fori_loop(0, B, per_batch, 0)

    @functools.partial(pl.run_scoped, exit_sem=pltpu.SemaphoreType.REGULAR)
    def _(exit_sem):
        pl.semaphore_signal(exit_sem, inc=1, device_id=nbr,
                            device_id_type=pl.DeviceIdType.MESH)
        pl.semaphore_wait(exit_sem, 1)


def kernel(Q, K, V, bt, lens):
    q2 = Q.reshape(B, HD)
    k2 = K.reshape(ROWS, HD)
    v2 = V.reshape(ROWS, HD)

    head_of_col = jnp.arange(HD, dtype=jnp.int32) // D
    head_mask = head_of_col[:, None] == jnp.arange(H, dtype=jnp.int32)[None]
    qmat = (q2[:, :, None] * head_mask[None]).astype(jnp.bfloat16)
    qmat = qmat.reshape(B * HD, H)

    out2 = pl.pallas_call(
        _body,
        out_shape=jax.ShapeDtypeStruct((B, HD), jnp.float32),
        in_specs=[
            pl.BlockSpec(memory_space=pltpu.VMEM),
            pl.BlockSpec(memory_space=pltpu.VMEM),
            pl.BlockSpec(memory_space=pltpu.VMEM),
            pl.BlockSpec(memory_space=pltpu.SMEM),
            pl.BlockSpec(memory_space=pltpu.SMEM),
        ],
        out_specs=pl.BlockSpec(memory_space=pltpu.VMEM),
        scratch_shapes=[
            pltpu.VMEM((2 * ROWS, HD), jnp.bfloat16),
            pltpu.VMEM((2 * ROWS, HD), jnp.bfloat16),
            pltpu.VMEM((NB * BS, HD), jnp.bfloat16),
            pltpu.VMEM((NB * BS, HD), jnp.bfloat16),
            pltpu.SemaphoreType.DMA((2,)),
            pltpu.SemaphoreType.DMA((2,)),
        ],
        compiler_params=pltpu.CompilerParams(collective_id=0),
    )(k2, v2, qmat, bt, lens)
    return out2.reshape(B, 1, H, D)
